# baseline (speedup 1.0000x reference)
import jax
import jax.numpy as jnp
from jax import lax
from jax.experimental import pallas as pl
from jax.experimental.pallas import tpu as pltpu, tpu_sc as plsc

B = 4 * 8192
D = 512
L = 16
NC = 2
NS = 16
NW = NC * NS
BPW = B // NW
NGRP = BPW // L
RPW = 32
REP_ROWS = NW * RPW * 2


def _sc_body(x_hbm, table_hbm, out_hbm, rep_hbm, idx_v, tbl_v, gsem, osem):
    wid = lax.axis_index("s") * NC + lax.axis_index("c")
    pltpu.sync_copy(x_hbm.at[wid], idx_v)
    pltpu.sync_copy(table_hbm, tbl_v)

    reps = []
    for r in range(RPW):
        c = pltpu.make_async_copy(
            tbl_v, rep_hbm.at[pl.ds((wid * RPW + r) * 2, 2)], gsem)
        c.start()
        reps.append(c)

    off0 = 2 * (wid * RPW) + 2 * lax.iota(jnp.int32, L)
    for g in range(NGRP):
        sl = pl.ds(g * L, L)
        idx_v[sl] = idx_v[sl] + (off0 + (g % 2) * 32)

    for c in reps:
        c.wait()

    base = wid * BPW

    def grpbody(g, carry):
        xv = idx_v[pl.ds(g * L, L)]
        row0 = base + g * L
        for l in range(L):
            pltpu.make_async_copy(
                rep_hbm.at[pl.ds(xv[l], 1)],
                out_hbm.at[pl.ds(row0 + l, 1)], osem).start()
        return carry

    lax.fori_loop(0, NGRP, grpbody, 0)

    def drainbody(g, carry):
        pltpu.make_async_copy(
            rep_hbm.at[pl.ds(0, 1)], out_hbm.at[pl.ds(0, 1)], osem).wait()
        return carry

    lax.fori_loop(0, BPW, drainbody, 0)


def kernel(x, table):
    xf = x.reshape(NW, BPW).astype(jnp.int32)
    out, _ = pl.kernel(
        _sc_body,
        out_type=[
            jax.ShapeDtypeStruct((B, D), jnp.float32),
            jax.ShapeDtypeStruct((REP_ROWS, D), jnp.float32),
        ],
        mesh=plsc.VectorSubcoreMesh(core_axis_name="c", subcore_axis_name="s"),
        scratch_types=[
            pltpu.VMEM((BPW,), jnp.int32),
            pltpu.VMEM((2, D), jnp.float32),
            pltpu.SemaphoreType.DMA,
            pltpu.SemaphoreType.DMA,
        ],
    )(xf, table)
    return out.reshape(x.shape[0], x.shape[1], D)


# speedup vs baseline: 27.1594x; 27.1594x over previous
import jax
import jax.numpy as jnp
from jax import lax
from jax.experimental import pallas as pl
from jax.experimental.pallas import tpu as pltpu, tpu_sc as plsc

B = 4 * 8192
D = 512
NC = 2
NS = 16
NW = NC * NS
BPW = B // NW
CHUNK = 64
NCHUNK = BPW // CHUNK
NBUF = 3
RPW = 32
REP_ROWS = NW * RPW * 2


def _sc_body(x_hbm, table_hbm, out_hbm, rep_hbm,
             idx_v, tbl_v, rows_v, gsem, osem):
    wid = lax.axis_index("s") * NC + lax.axis_index("c")
    pltpu.sync_copy(x_hbm.at[wid], idx_v)
    pltpu.sync_copy(table_hbm, tbl_v)

    reps = []
    for r in range(RPW):
        c = pltpu.make_async_copy(
            tbl_v, rep_hbm.at[pl.ds((wid * RPW + r) * 2, 2)], osem)
        c.start()
        reps.append(c)

    off0 = 2 * (wid * RPW) + 2 * lax.iota(jnp.int32, 16)
    for c16 in range(NCHUNK):
        for g in range(CHUNK // 16):
            sl = pl.ds(g * 16, 16)
            idx_v[c16, sl] = idx_v[c16, sl] + (off0 + (g % 2) * 32)

    for c in reps:
        c.wait()

    base = wid * BPW
    gathers = [None] * NCHUNK
    outs = [None] * NCHUNK
    for j in range(min(NBUF, NCHUNK)):
        gathers[j] = pltpu.make_async_copy(
            rep_hbm.at[idx_v.at[j]], rows_v.at[j % NBUF], gsem)
        gathers[j].start()
    for j in range(NCHUNK):
        b = j % NBUF
        gathers[j].wait()
        outs[j] = pltpu.make_async_copy(
            rows_v.at[b], out_hbm.at[pl.ds(base + j * CHUNK, CHUNK)], osem)
        outs[j].start()
        nj = j + NBUF
        if nj < NCHUNK:
            outs[j].wait()
            gathers[nj] = pltpu.make_async_copy(
                rep_hbm.at[idx_v.at[nj]], rows_v.at[b], gsem)
            gathers[nj].start()
    for j in range(max(0, NCHUNK - NBUF), NCHUNK):
        outs[j].wait()


def kernel(x, table):
    xf = x.reshape(NW, NCHUNK, CHUNK).astype(jnp.int32)
    out, _ = pl.kernel(
        _sc_body,
        out_type=[
            jax.ShapeDtypeStruct((B, D), jnp.float32),
            jax.ShapeDtypeStruct((REP_ROWS, D), jnp.float32),
        ],
        mesh=plsc.VectorSubcoreMesh(core_axis_name="c", subcore_axis_name="s"),
        scratch_types=[
            pltpu.VMEM((NCHUNK, CHUNK), jnp.int32),
            pltpu.VMEM((2, D), jnp.float32),
            pltpu.VMEM((NBUF, CHUNK, D), jnp.float32),
            pltpu.SemaphoreType.DMA,
            pltpu.SemaphoreType.DMA,
        ],
    )(xf, table)
    return out.reshape(x.shape[0], x.shape[1], D)
